# baseline (speedup 1.0000x reference)
import functools

import jax
import jax.numpy as jnp
import numpy as np
from jax.experimental import pallas as pl
from jax.experimental.pallas import tpu as pltpu

_TINY = np.float32(np.finfo(np.float32).tiny)
_KS0 = 0
_KS1 = 42
_KS2 = _KS0 ^ _KS1 ^ 0x1BD11BDA

_ROTS_A = (13, 15, 26, 6)
_ROTS_B = (17, 29, 16, 24)
_INJECT = (
    (_KS1, _KS2, 1),
    (_KS2, _KS0, 2),
    (_KS0, _KS1, 3),
    (_KS1, _KS2, 4),
    (_KS2, _KS0, 5),
)


def _rotl(x, d):
    return (x << jnp.uint32(d)) | (x >> jnp.uint32(32 - d))


def _threefry_bits(idx_u32):
    x0 = jnp.full_like(idx_u32, jnp.uint32(_KS0))
    x1 = idx_u32 + jnp.uint32(_KS1)
    for group in range(5):
        rots = _ROTS_A if group % 2 == 0 else _ROTS_B
        for r in rots:
            x0 = x0 + x1
            x1 = _rotl(x1, r)
            x1 = x1 ^ x0
        a, b, c = _INJECT[group]
        x0 = x0 + jnp.uint32(a)
        x1 = x1 + jnp.uint32(b + c)
    return x0 ^ x1


def _gumbel_from_bits(bits):
    float_bits = (bits >> jnp.uint32(9)) | jnp.uint32(0x3F800000)
    frac = jax.lax.bitcast_convert_type(float_bits, jnp.float32) - jnp.float32(1.0)
    u = jnp.maximum(_TINY, frac * jnp.float32(1.0 - float(_TINY)) + _TINY)
    return -jnp.log(-jnp.log(u))


def _sample_kernel(logp_ref, out_ref, acc_val, acc_idx, *, n_cols, r_blk, c_blk,
                   n_cblocks):
    r = pl.program_id(0)
    j = pl.program_id(1)

    row = jax.lax.broadcasted_iota(jnp.uint32, (r_blk, c_blk), 0) + (
        r.astype(jnp.uint32) * jnp.uint32(r_blk))
    col = jax.lax.broadcasted_iota(jnp.uint32, (r_blk, c_blk), 1) + (
        j.astype(jnp.uint32) * jnp.uint32(c_blk))
    flat = row * jnp.uint32(n_cols) + col

    g = _gumbel_from_bits(_threefry_bits(flat))
    score = logp_ref[...] + g
    if n_cols % c_blk != 0:
        score = jnp.where(col < jnp.uint32(n_cols), score,
                          jnp.float32(-jnp.inf))
    idx = flat.astype(jnp.int32)

    @pl.when(j == 0)
    def _init():
        acc_val[...] = score
        acc_idx[...] = idx

    @pl.when(j > 0)
    def _update():
        prev_val = acc_val[...]
        prev_idx = acc_idx[...]
        take = score > prev_val
        acc_val[...] = jnp.where(take, score, prev_val)
        acc_idx[...] = jnp.where(take, idx, prev_idx)

    @pl.when(j == n_cblocks - 1)
    def _finalize():
        val = acc_val[...]
        ind = acc_idx[...]
        row_max = jnp.max(val, axis=1, keepdims=True)
        cand = jnp.where(val >= row_max, ind, jnp.int32(2**31 - 1))
        best = jnp.min(cand, axis=1, keepdims=True)
        out_ref[...] = jnp.broadcast_to(best, (r_blk, 128))


@functools.partial(jax.jit, static_argnames=())
def kernel(log_p):
    n_rows, n_cols = log_p.shape
    r_blk = min(n_rows, 16)
    assert n_rows % r_blk == 0
    c_blk = 2048
    n_cblocks = pl.cdiv(n_cols, c_blk)
    n_rblocks = n_rows // r_blk

    body = functools.partial(
        _sample_kernel, n_cols=n_cols, r_blk=r_blk, c_blk=c_blk,
        n_cblocks=n_cblocks)

    out = pl.pallas_call(
        body,
        grid=(n_rblocks, n_cblocks),
        in_specs=[pl.BlockSpec((r_blk, c_blk), lambda r, j: (r, j))],
        out_specs=pl.BlockSpec((r_blk, 128), lambda r, j: (r, 0)),
        out_shape=jax.ShapeDtypeStruct((n_rows, 128), jnp.int32),
        scratch_shapes=[
            pltpu.VMEM((r_blk, c_blk), jnp.float32),
            pltpu.VMEM((r_blk, c_blk), jnp.int32),
        ],
        compiler_params=pltpu.CompilerParams(
            dimension_semantics=("parallel", "arbitrary"),
        ),
    )(log_p)

    samples = out[:, 0] - jnp.arange(n_rows, dtype=jnp.int32) * jnp.int32(n_cols)
    return samples.astype(jnp.int64)


# speedup vs baseline: 1.0771x; 1.0771x over previous
import functools

import jax
import jax.numpy as jnp
import numpy as np
from jax.experimental import pallas as pl
from jax.experimental.pallas import tpu as pltpu

_TINY = np.float32(np.finfo(np.float32).tiny)
_KS0 = np.uint32(0)
_KS1 = np.uint32(42)
_KS2 = np.uint32(42 ^ 0x1BD11BDA)

_C_BLK = 1024
_SLICE = 128


def _rotl(x, d):
    return (x << jnp.uint32(d)) | (x >> jnp.uint32(32 - d))


def _bits_from_x1(x):
    x0 = x
    x1 = _rotl(x, 13) ^ x
    for r in (15, 26, 6):
        x0 = x0 + x1
        x1 = _rotl(x1, r) ^ x0
    x0 = x0 + _KS1
    x1 = x1 + jnp.uint32(_KS2 + np.uint32(1))
    for r in (17, 29, 16, 24):
        x0 = x0 + x1
        x1 = _rotl(x1, r) ^ x0
    x0 = x0 + _KS2
    x1 = x1 + jnp.uint32(2)
    for r in (13, 15, 26, 6):
        x0 = x0 + x1
        x1 = _rotl(x1, r) ^ x0
    x1 = x1 + jnp.uint32(_KS1 + np.uint32(3))
    for r in (17, 29, 16, 24):
        x0 = x0 + x1
        x1 = _rotl(x1, r) ^ x0
    x0 = x0 + _KS1
    x1 = x1 + jnp.uint32(_KS2 + np.uint32(4))
    for r in (13, 15, 26, 6):
        x0 = x0 + x1
        x1 = _rotl(x1, r) ^ x0
    x0 = x0 + _KS2
    x1 = x1 + jnp.uint32(5)
    return x0 ^ x1


def _score_slice(lp, x1_in):
    bits = _bits_from_x1(x1_in)
    float_bits = (bits >> jnp.uint32(9)) | jnp.uint32(0x3F800000)
    frac = jax.lax.bitcast_convert_type(float_bits, jnp.float32) - jnp.float32(1.0)
    u = jnp.maximum(frac, _TINY)
    g = -jnp.log(-jnp.log(u))
    return lp + g


def _sample_kernel(logp_ref, tail_ref, out_ref, acc_val, acc_idx, *, n_rows,
                   n_cols, n_blocks):
    j = pl.program_id(0)
    n_slices = _C_BLK // _SLICE

    pattern = (
        jax.lax.broadcasted_iota(jnp.uint32, (n_rows, _SLICE), 0)
        * jnp.uint32(n_cols)
        + jax.lax.broadcasted_iota(jnp.uint32, (n_rows, _SLICE), 1)
        + _KS1
    )

    @pl.when(j == 0)
    def _init():
        acc_val[...] = jnp.full((n_rows, _SLICE), -jnp.inf, jnp.float32)
        acc_idx[...] = jnp.zeros((n_rows, _SLICE), jnp.int32)

    v = acc_val[...]
    ix = acc_idx[...]
    base = j.astype(jnp.uint32) * jnp.uint32(_C_BLK)
    for s in range(n_slices):
        x1_in = pattern + (base + jnp.uint32(s * _SLICE))
        score = _score_slice(logp_ref[:, s * _SLICE:(s + 1) * _SLICE], x1_in)
        idx = x1_in.astype(jnp.int32)
        take = score > v
        v = jnp.where(take, score, v)
        ix = jnp.where(take, idx, ix)

    @pl.when(j < n_blocks - 1)
    def _store():
        acc_val[...] = v
        acc_idx[...] = ix

    @pl.when(j == n_blocks - 1)
    def _tail_and_finalize():
        vv, iixx = v, ix
        tail_base = jnp.uint32(n_blocks * _C_BLK)
        for s in range(n_slices):
            x1_in = pattern + (tail_base + jnp.uint32(s * _SLICE))
            score = _score_slice(tail_ref[:, s * _SLICE:(s + 1) * _SLICE],
                                 x1_in)
            idx = x1_in.astype(jnp.int32)
            take = score > vv
            vv = jnp.where(take, score, vv)
            iixx = jnp.where(take, idx, iixx)
        row_max = jnp.max(vv, axis=1, keepdims=True)
        cand = jnp.where(vv >= row_max, iixx, jnp.int32(2**31 - 1))
        best = jnp.min(cand, axis=1, keepdims=True)
        out_ref[...] = jnp.broadcast_to(best - jnp.int32(int(_KS1)),
                                        (n_rows, _SLICE))


@jax.jit
def kernel(log_p):
    n_rows, n_cols = log_p.shape
    n_blocks = n_cols // _C_BLK
    tail_cols = n_cols - n_blocks * _C_BLK

    tail = jnp.full((n_rows, _C_BLK), -jnp.inf, jnp.float32)
    if tail_cols:
        tail = jax.lax.dynamic_update_slice(
            tail, log_p[:, n_blocks * _C_BLK:], (0, 0))

    body = functools.partial(
        _sample_kernel, n_rows=n_rows, n_cols=n_cols, n_blocks=n_blocks)

    out = pl.pallas_call(
        body,
        grid=(n_blocks,),
        in_specs=[
            pl.BlockSpec((n_rows, _C_BLK), lambda j: (0, j)),
            pl.BlockSpec((n_rows, _C_BLK), lambda j: (0, 0)),
        ],
        out_specs=pl.BlockSpec((n_rows, _SLICE), lambda j: (0, 0)),
        out_shape=jax.ShapeDtypeStruct((n_rows, _SLICE), jnp.int32),
        scratch_shapes=[
            pltpu.VMEM((n_rows, _SLICE), jnp.float32),
            pltpu.VMEM((n_rows, _SLICE), jnp.int32),
        ],
        compiler_params=pltpu.CompilerParams(
            dimension_semantics=("arbitrary",),
        ),
    )(log_p, tail)

    samples = out[:, 0] - jnp.arange(n_rows, dtype=jnp.int32) * jnp.int32(n_cols)
    return samples.astype(jnp.int64)


# speedup vs baseline: 1.4667x; 1.3617x over previous
import functools

import jax
import jax.numpy as jnp
import numpy as np
from jax.experimental import pallas as pl
from jax.experimental.pallas import tpu as pltpu

_TINY = np.float32(np.finfo(np.float32).tiny)
_KS0 = np.uint32(0)
_KS1 = np.uint32(42)
_KS2 = np.uint32(42 ^ 0x1BD11BDA)

_C_BLK = 2048
_SLICE = 128


def _rotl(x, d):
    return (x << jnp.uint32(d)) | (x >> jnp.uint32(32 - d))


def _bits_from_x1(x):
    x0 = x
    x1 = _rotl(x, 13) ^ x
    for r in (15, 26, 6):
        x0 = x0 + x1
        x1 = _rotl(x1, r) ^ x0
    x0 = x0 + _KS1
    x1 = x1 + jnp.uint32(_KS2 + np.uint32(1))
    for r in (17, 29, 16, 24):
        x0 = x0 + x1
        x1 = _rotl(x1, r) ^ x0
    x0 = x0 + _KS2
    x1 = x1 + jnp.uint32(2)
    for r in (13, 15, 26, 6):
        x0 = x0 + x1
        x1 = _rotl(x1, r) ^ x0
    x1 = x1 + jnp.uint32(_KS1 + np.uint32(3))
    for r in (17, 29, 16, 24):
        x0 = x0 + x1
        x1 = _rotl(x1, r) ^ x0
    x0 = x0 + _KS1
    x1 = x1 + jnp.uint32(_KS2 + np.uint32(4))
    for r in (13, 15, 26, 6):
        x0 = x0 + x1
        x1 = _rotl(x1, r) ^ x0
    x0 = x0 + _KS2
    x1 = x1 + jnp.uint32(5)
    return x0 ^ x1


def _score_slice(lp, x1_in):
    bits = _bits_from_x1(x1_in)
    float_bits = (bits >> jnp.uint32(9)) | jnp.uint32(0x3F800000)
    frac = jax.lax.bitcast_convert_type(float_bits, jnp.float32) - jnp.float32(1.0)
    u = jnp.maximum(frac, _TINY)
    g = -jnp.log(-jnp.log(u))
    return lp + g


def _sample_kernel(logp_ref, tail_ref, out_ref, acc_val, acc_idx, *, n_rows,
                   n_cols, n_blocks, tail_blk):
    j = pl.program_id(0)
    n_slices = _C_BLK // _SLICE
    n_tail_slices = tail_blk // _SLICE

    pattern = (
        jax.lax.broadcasted_iota(jnp.uint32, (n_rows, _SLICE), 0)
        * jnp.uint32(n_cols)
        + jax.lax.broadcasted_iota(jnp.uint32, (n_rows, _SLICE), 1)
        + _KS1
    )

    @pl.when(j == 0)
    def _init():
        acc_val[...] = jnp.full((n_rows, _SLICE), -jnp.inf, jnp.float32)
        acc_idx[...] = jnp.zeros((n_rows, _SLICE), jnp.int32)

    v = acc_val[...]
    ix = acc_idx[...]
    base = j.astype(jnp.uint32) * jnp.uint32(_C_BLK)
    for s in range(n_slices):
        x1_in = pattern + (base + jnp.uint32(s * _SLICE))
        score = _score_slice(logp_ref[:, s * _SLICE:(s + 1) * _SLICE], x1_in)
        idx = x1_in.astype(jnp.int32)
        take = score > v
        v = jnp.where(take, score, v)
        ix = jnp.where(take, idx, ix)

    @pl.when(j < n_blocks - 1)
    def _store():
        acc_val[...] = v
        acc_idx[...] = ix

    @pl.when(j == n_blocks - 1)
    def _tail_and_finalize():
        vv, iixx = v, ix
        tail_base = jnp.uint32(n_blocks * _C_BLK)
        for s in range(n_tail_slices):
            x1_in = pattern + (tail_base + jnp.uint32(s * _SLICE))
            score = _score_slice(tail_ref[:, s * _SLICE:(s + 1) * _SLICE],
                                 x1_in)
            idx = x1_in.astype(jnp.int32)
            take = score > vv
            vv = jnp.where(take, score, vv)
            iixx = jnp.where(take, idx, iixx)
        row_max = jnp.max(vv, axis=1, keepdims=True)
        cand = jnp.where(vv >= row_max, iixx, jnp.int32(2**31 - 1))
        best = jnp.min(cand, axis=1, keepdims=True)
        out_ref[...] = jnp.broadcast_to(best - jnp.int32(int(_KS1)),
                                        (n_rows, _SLICE))


@jax.jit
def kernel(log_p):
    n_rows, n_cols = log_p.shape
    n_blocks = n_cols // _C_BLK
    tail_cols = n_cols - n_blocks * _C_BLK
    tail_blk = max(_SLICE, pl.cdiv(tail_cols, _SLICE) * _SLICE)

    tail = jnp.full((n_rows, tail_blk), -jnp.inf, jnp.float32)
    if tail_cols:
        tail = jax.lax.dynamic_update_slice(
            tail, log_p[:, n_blocks * _C_BLK:], (0, 0))

    body = functools.partial(
        _sample_kernel, n_rows=n_rows, n_cols=n_cols, n_blocks=n_blocks,
        tail_blk=tail_blk)

    out = pl.pallas_call(
        body,
        grid=(n_blocks,),
        in_specs=[
            pl.BlockSpec((n_rows, _C_BLK), lambda j: (0, j)),
            pl.BlockSpec((n_rows, tail_blk), lambda j: (0, 0)),
        ],
        out_specs=pl.BlockSpec((n_rows, _SLICE), lambda j: (0, 0)),
        out_shape=jax.ShapeDtypeStruct((n_rows, _SLICE), jnp.int32),
        scratch_shapes=[
            pltpu.VMEM((n_rows, _SLICE), jnp.float32),
            pltpu.VMEM((n_rows, _SLICE), jnp.int32),
        ],
        compiler_params=pltpu.CompilerParams(
            dimension_semantics=("arbitrary",),
        ),
    )(log_p, tail)

    samples = out[:, 0] - jnp.arange(n_rows, dtype=jnp.int32) * jnp.int32(n_cols)
    return samples.astype(jnp.int64)


# speedup vs baseline: 1.4978x; 1.0212x over previous
import functools

import jax
import jax.numpy as jnp
import numpy as np
from jax.experimental import pallas as pl
from jax.experimental.pallas import tpu as pltpu

_TINY = np.float32(np.finfo(np.float32).tiny)
_KS0 = np.uint32(0)
_KS1 = np.uint32(42)
_KS2 = np.uint32(42 ^ 0x1BD11BDA)

_C_BLK = 4096
_SLICE = 128


def _rotl(x, d):
    return (x << jnp.uint32(d)) | (x >> jnp.uint32(32 - d))


def _bits_from_x1(x):
    x0 = x
    x1 = _rotl(x, 13) ^ x
    for r in (15, 26, 6):
        x0 = x0 + x1
        x1 = _rotl(x1, r) ^ x0
    x0 = x0 + _KS1
    x1 = x1 + jnp.uint32(_KS2 + np.uint32(1))
    for r in (17, 29, 16, 24):
        x0 = x0 + x1
        x1 = _rotl(x1, r) ^ x0
    x0 = x0 + _KS2
    x1 = x1 + jnp.uint32(2)
    for r in (13, 15, 26, 6):
        x0 = x0 + x1
        x1 = _rotl(x1, r) ^ x0
    x1 = x1 + jnp.uint32(_KS1 + np.uint32(3))
    for r in (17, 29, 16, 24):
        x0 = x0 + x1
        x1 = _rotl(x1, r) ^ x0
    x0 = x0 + _KS1
    x1 = x1 + jnp.uint32(_KS2 + np.uint32(4))
    for r in (13, 15, 26, 6):
        x0 = x0 + x1
        x1 = _rotl(x1, r) ^ x0
    x0 = x0 + _KS2
    x1 = x1 + jnp.uint32(5)
    return x0 ^ x1


def _score_slice(lp, x1_in):
    bits = _bits_from_x1(x1_in)
    float_bits = (bits >> jnp.uint32(9)) | jnp.uint32(0x3F800000)
    frac = jax.lax.bitcast_convert_type(float_bits, jnp.float32) - jnp.float32(1.0)
    u = jnp.maximum(frac, _TINY)
    g = -jnp.log(-jnp.log(u))
    return lp + g


def _sample_kernel(logp_ref, tail_ref, out_ref, acc_val, acc_idx, *, n_rows,
                   n_cols, n_blocks, tail_blk):
    j = pl.program_id(0)
    n_slices = _C_BLK // _SLICE
    n_tail_slices = tail_blk // _SLICE

    pattern = (
        jax.lax.broadcasted_iota(jnp.uint32, (n_rows, _SLICE), 0)
        * jnp.uint32(n_cols)
        + jax.lax.broadcasted_iota(jnp.uint32, (n_rows, _SLICE), 1)
        + _KS1
    )

    @pl.when(j == 0)
    def _init():
        acc_val[...] = jnp.full((n_rows, _SLICE), -jnp.inf, jnp.float32)
        acc_idx[...] = jnp.zeros((n_rows, _SLICE), jnp.int32)

    v = acc_val[...]
    ix = acc_idx[...]
    base = j.astype(jnp.uint32) * jnp.uint32(_C_BLK)
    for s in range(n_slices):
        x1_in = pattern + (base + jnp.uint32(s * _SLICE))
        score = _score_slice(logp_ref[:, s * _SLICE:(s + 1) * _SLICE], x1_in)
        idx = x1_in.astype(jnp.int32)
        take = score > v
        v = jnp.where(take, score, v)
        ix = jnp.where(take, idx, ix)

    @pl.when(j < n_blocks - 1)
    def _store():
        acc_val[...] = v
        acc_idx[...] = ix

    @pl.when(j == n_blocks - 1)
    def _tail_and_finalize():
        vv, iixx = v, ix
        tail_base = jnp.uint32(n_blocks * _C_BLK)
        for s in range(n_tail_slices):
            x1_in = pattern + (tail_base + jnp.uint32(s * _SLICE))
            score = _score_slice(tail_ref[:, s * _SLICE:(s + 1) * _SLICE],
                                 x1_in)
            idx = x1_in.astype(jnp.int32)
            take = score > vv
            vv = jnp.where(take, score, vv)
            iixx = jnp.where(take, idx, iixx)
        row_max = jnp.max(vv, axis=1, keepdims=True)
        cand = jnp.where(vv >= row_max, iixx, jnp.int32(2**31 - 1))
        best = jnp.min(cand, axis=1, keepdims=True)
        out_ref[...] = jnp.broadcast_to(best - jnp.int32(int(_KS1)),
                                        (n_rows, _SLICE))


@jax.jit
def kernel(log_p):
    n_rows, n_cols = log_p.shape
    n_blocks = n_cols // _C_BLK
    tail_cols = n_cols - n_blocks * _C_BLK
    tail_blk = max(_SLICE, pl.cdiv(tail_cols, _SLICE) * _SLICE)

    tail = jnp.full((n_rows, tail_blk), -jnp.inf, jnp.float32)
    if tail_cols:
        tail = jax.lax.dynamic_update_slice(
            tail, log_p[:, n_blocks * _C_BLK:], (0, 0))

    body = functools.partial(
        _sample_kernel, n_rows=n_rows, n_cols=n_cols, n_blocks=n_blocks,
        tail_blk=tail_blk)

    out = pl.pallas_call(
        body,
        grid=(n_blocks,),
        in_specs=[
            pl.BlockSpec((n_rows, _C_BLK), lambda j: (0, j)),
            pl.BlockSpec((n_rows, tail_blk), lambda j: (0, 0)),
        ],
        out_specs=pl.BlockSpec((n_rows, _SLICE), lambda j: (0, 0)),
        out_shape=jax.ShapeDtypeStruct((n_rows, _SLICE), jnp.int32),
        scratch_shapes=[
            pltpu.VMEM((n_rows, _SLICE), jnp.float32),
            pltpu.VMEM((n_rows, _SLICE), jnp.int32),
        ],
        compiler_params=pltpu.CompilerParams(
            dimension_semantics=("arbitrary",),
        ),
    )(log_p, tail)

    samples = out[:, 0] - jnp.arange(n_rows, dtype=jnp.int32) * jnp.int32(n_cols)
    return samples.astype(jnp.int64)


# speedup vs baseline: 1.4989x; 1.0007x over previous
import functools

import jax
import jax.numpy as jnp
import numpy as np
from jax.experimental import pallas as pl
from jax.experimental.pallas import tpu as pltpu

_TINY = np.float32(np.finfo(np.float32).tiny)
_KS0 = np.uint32(0)
_KS1 = np.uint32(42)
_KS2 = np.uint32(42 ^ 0x1BD11BDA)

_C_BLK = 4096
_SLICE = 128


def _rotl(x, d):
    return (x << jnp.uint32(d)) | (x >> jnp.uint32(32 - d))


def _bits_from_x1(x):
    x0 = x
    x1 = _rotl(x, 13) ^ x
    for r in (15, 26, 6):
        x0 = x0 + x1
        x1 = _rotl(x1, r) ^ x0
    x0 = x0 + _KS1
    x1 = x1 + jnp.uint32(_KS2 + np.uint32(1))
    for r in (17, 29, 16, 24):
        x0 = x0 + x1
        x1 = _rotl(x1, r) ^ x0
    x0 = x0 + _KS2
    x1 = x1 + jnp.uint32(2)
    for r in (13, 15, 26, 6):
        x0 = x0 + x1
        x1 = _rotl(x1, r) ^ x0
    x1 = x1 + jnp.uint32(_KS1 + np.uint32(3))
    for r in (17, 29, 16, 24):
        x0 = x0 + x1
        x1 = _rotl(x1, r) ^ x0
    x0 = x0 + _KS1
    x1 = x1 + jnp.uint32(_KS2 + np.uint32(4))
    for r in (13, 15, 26, 6):
        x0 = x0 + x1
        x1 = _rotl(x1, r) ^ x0
    x0 = x0 + _KS2
    x1 = x1 + jnp.uint32(5)
    return x0 ^ x1


def _score_slice(lp, x1_in):
    bits = _bits_from_x1(x1_in)
    float_bits = (bits >> jnp.uint32(9)) | jnp.uint32(0x3F800000)
    frac = jax.lax.bitcast_convert_type(float_bits, jnp.float32) - jnp.float32(1.0)
    u = jnp.maximum(frac, _TINY)
    g = -jnp.log(-jnp.log(u))
    return lp + g


def _sample_kernel(logp_ref, tail_ref, out_ref, acc_val, acc_idx, *, n_rows,
                   n_cols, n_blocks, tail_blk):
    j = pl.program_id(0)
    n_slices = _C_BLK // _SLICE
    n_tail_slices = tail_blk // _SLICE

    pattern = (
        jax.lax.broadcasted_iota(jnp.uint32, (n_rows, _SLICE), 0)
        * jnp.uint32(n_cols)
        + jax.lax.broadcasted_iota(jnp.uint32, (n_rows, _SLICE), 1)
        + _KS1
    )

    @pl.when(j == 0)
    def _init():
        acc_val[...] = jnp.full((n_rows, _SLICE), -jnp.inf, jnp.float32)
        acc_idx[...] = jnp.zeros((n_rows, _SLICE), jnp.int32)

    base = j.astype(jnp.uint32) * jnp.uint32(_C_BLK)

    n_chains = 4
    per_chain = n_slices // n_chains
    chains = []
    for c in range(n_chains):
        cv = ci = None
        for s in range(c * per_chain, (c + 1) * per_chain):
            x1_in = pattern + (base + jnp.uint32(s * _SLICE))
            score = _score_slice(logp_ref[:, s * _SLICE:(s + 1) * _SLICE],
                                 x1_in)
            idx = x1_in.astype(jnp.int32)
            if cv is None:
                cv, ci = score, idx
            else:
                take = score > cv
                cv = jnp.where(take, score, cv)
                ci = jnp.where(take, idx, ci)
        chains.append((cv, ci))

    v = acc_val[...]
    ix = acc_idx[...]
    for cv, ci in chains:
        take = cv > v
        v = jnp.where(take, cv, v)
        ix = jnp.where(take, ci, ix)

    @pl.when(j < n_blocks - 1)
    def _store():
        acc_val[...] = v
        acc_idx[...] = ix

    @pl.when(j == n_blocks - 1)
    def _tail_and_finalize():
        vv, iixx = v, ix
        tail_base = jnp.uint32(n_blocks * _C_BLK)
        for s in range(n_tail_slices):
            x1_in = pattern + (tail_base + jnp.uint32(s * _SLICE))
            score = _score_slice(tail_ref[:, s * _SLICE:(s + 1) * _SLICE],
                                 x1_in)
            idx = x1_in.astype(jnp.int32)
            take = score > vv
            vv = jnp.where(take, score, vv)
            iixx = jnp.where(take, idx, iixx)
        row_max = jnp.max(vv, axis=1, keepdims=True)
        cand = jnp.where(vv >= row_max, iixx, jnp.int32(2**31 - 1))
        best = jnp.min(cand, axis=1, keepdims=True)
        out_ref[...] = jnp.broadcast_to(best - jnp.int32(int(_KS1)),
                                        (n_rows, _SLICE))


@jax.jit
def kernel(log_p):
    n_rows, n_cols = log_p.shape
    n_blocks = n_cols // _C_BLK
    tail_cols = n_cols - n_blocks * _C_BLK
    tail_blk = max(_SLICE, pl.cdiv(tail_cols, _SLICE) * _SLICE)

    tail = jnp.full((n_rows, tail_blk), -jnp.inf, jnp.float32)
    if tail_cols:
        tail = jax.lax.dynamic_update_slice(
            tail, log_p[:, n_blocks * _C_BLK:], (0, 0))

    body = functools.partial(
        _sample_kernel, n_rows=n_rows, n_cols=n_cols, n_blocks=n_blocks,
        tail_blk=tail_blk)

    out = pl.pallas_call(
        body,
        grid=(n_blocks,),
        in_specs=[
            pl.BlockSpec((n_rows, _C_BLK), lambda j: (0, j)),
            pl.BlockSpec((n_rows, tail_blk), lambda j: (0, 0)),
        ],
        out_specs=pl.BlockSpec((n_rows, _SLICE), lambda j: (0, 0)),
        out_shape=jax.ShapeDtypeStruct((n_rows, _SLICE), jnp.int32),
        scratch_shapes=[
            pltpu.VMEM((n_rows, _SLICE), jnp.float32),
            pltpu.VMEM((n_rows, _SLICE), jnp.int32),
        ],
        compiler_params=pltpu.CompilerParams(
            dimension_semantics=("arbitrary",),
        ),
    )(log_p, tail)

    samples = out[:, 0] - jnp.arange(n_rows, dtype=jnp.int32) * jnp.int32(n_cols)
    return samples.astype(jnp.int64)


# speedup vs baseline: 1.5142x; 1.0101x over previous
import functools

import jax
import jax.numpy as jnp
import numpy as np
from jax.experimental import pallas as pl
from jax.experimental.pallas import tpu as pltpu

_TINY = np.float32(np.finfo(np.float32).tiny)
_KS0 = np.uint32(0)
_KS1 = np.uint32(42)
_KS2 = np.uint32(42 ^ 0x1BD11BDA)

_C_BLK = 8192
_SLICE = 128


def _rotl(x, d):
    return (x << jnp.uint32(d)) | (x >> jnp.uint32(32 - d))


def _bits_from_x1(x):
    x0 = x
    x1 = _rotl(x, 13) ^ x
    for r in (15, 26, 6):
        x0 = x0 + x1
        x1 = _rotl(x1, r) ^ x0
    x0 = x0 + _KS1
    x1 = x1 + jnp.uint32(_KS2 + np.uint32(1))
    for r in (17, 29, 16, 24):
        x0 = x0 + x1
        x1 = _rotl(x1, r) ^ x0
    x0 = x0 + _KS2
    x1 = x1 + jnp.uint32(2)
    for r in (13, 15, 26, 6):
        x0 = x0 + x1
        x1 = _rotl(x1, r) ^ x0
    x1 = x1 + jnp.uint32(_KS1 + np.uint32(3))
    for r in (17, 29, 16, 24):
        x0 = x0 + x1
        x1 = _rotl(x1, r) ^ x0
    x0 = x0 + _KS1
    x1 = x1 + jnp.uint32(_KS2 + np.uint32(4))
    for r in (13, 15, 26, 6):
        x0 = x0 + x1
        x1 = _rotl(x1, r) ^ x0
    x0 = x0 + _KS2
    x1 = x1 + jnp.uint32(5)
    return x0 ^ x1


def _score_slice(lp, x1_in):
    bits = _bits_from_x1(x1_in)
    float_bits = (bits >> jnp.uint32(9)) | jnp.uint32(0x3F800000)
    frac = jax.lax.bitcast_convert_type(float_bits, jnp.float32) - jnp.float32(1.0)
    u = jnp.maximum(frac, _TINY)
    g = -jnp.log(-jnp.log(u))
    return lp + g


def _sample_kernel(logp_ref, tail_ref, out_ref, acc_val, acc_idx, *, n_rows,
                   n_cols, n_blocks, tail_blk):
    j = pl.program_id(0)
    n_slices = _C_BLK // _SLICE
    n_tail_slices = tail_blk // _SLICE

    pattern = (
        jax.lax.broadcasted_iota(jnp.uint32, (n_rows, _SLICE), 0)
        * jnp.uint32(n_cols)
        + jax.lax.broadcasted_iota(jnp.uint32, (n_rows, _SLICE), 1)
        + _KS1
    )

    @pl.when(j == 0)
    def _init():
        acc_val[...] = jnp.full((n_rows, _SLICE), -jnp.inf, jnp.float32)
        acc_idx[...] = jnp.zeros((n_rows, _SLICE), jnp.int32)

    base = j.astype(jnp.uint32) * jnp.uint32(_C_BLK)

    n_chains = 4
    per_chain = n_slices // n_chains
    chains = []
    for c in range(n_chains):
        cv = ci = None
        for s in range(c * per_chain, (c + 1) * per_chain):
            x1_in = pattern + (base + jnp.uint32(s * _SLICE))
            score = _score_slice(logp_ref[:, s * _SLICE:(s + 1) * _SLICE],
                                 x1_in)
            idx = x1_in.astype(jnp.int32)
            if cv is None:
                cv, ci = score, idx
            else:
                take = score > cv
                cv = jnp.where(take, score, cv)
                ci = jnp.where(take, idx, ci)
        chains.append((cv, ci))

    v = acc_val[...]
    ix = acc_idx[...]
    for cv, ci in chains:
        take = cv > v
        v = jnp.where(take, cv, v)
        ix = jnp.where(take, ci, ix)

    @pl.when(j < n_blocks - 1)
    def _store():
        acc_val[...] = v
        acc_idx[...] = ix

    @pl.when(j == n_blocks - 1)
    def _tail_and_finalize():
        vv, iixx = v, ix
        tail_base = jnp.uint32(n_blocks * _C_BLK)
        for s in range(n_tail_slices):
            x1_in = pattern + (tail_base + jnp.uint32(s * _SLICE))
            score = _score_slice(tail_ref[:, s * _SLICE:(s + 1) * _SLICE],
                                 x1_in)
            idx = x1_in.astype(jnp.int32)
            take = score > vv
            vv = jnp.where(take, score, vv)
            iixx = jnp.where(take, idx, iixx)
        row_max = jnp.max(vv, axis=1, keepdims=True)
        cand = jnp.where(vv >= row_max, iixx, jnp.int32(2**31 - 1))
        best = jnp.min(cand, axis=1, keepdims=True)
        out_ref[...] = jnp.broadcast_to(best - jnp.int32(int(_KS1)),
                                        (n_rows, _SLICE))


@jax.jit
def kernel(log_p):
    n_rows, n_cols = log_p.shape
    n_blocks = n_cols // _C_BLK
    tail_cols = n_cols - n_blocks * _C_BLK
    tail_blk = max(_SLICE, pl.cdiv(tail_cols, _SLICE) * _SLICE)

    tail = jnp.full((n_rows, tail_blk), -jnp.inf, jnp.float32)
    if tail_cols:
        tail = jax.lax.dynamic_update_slice(
            tail, log_p[:, n_blocks * _C_BLK:], (0, 0))

    body = functools.partial(
        _sample_kernel, n_rows=n_rows, n_cols=n_cols, n_blocks=n_blocks,
        tail_blk=tail_blk)

    out = pl.pallas_call(
        body,
        grid=(n_blocks,),
        in_specs=[
            pl.BlockSpec((n_rows, _C_BLK), lambda j: (0, j)),
            pl.BlockSpec((n_rows, tail_blk), lambda j: (0, 0)),
        ],
        out_specs=pl.BlockSpec((n_rows, _SLICE), lambda j: (0, 0)),
        out_shape=jax.ShapeDtypeStruct((n_rows, _SLICE), jnp.int32),
        scratch_shapes=[
            pltpu.VMEM((n_rows, _SLICE), jnp.float32),
            pltpu.VMEM((n_rows, _SLICE), jnp.int32),
        ],
        compiler_params=pltpu.CompilerParams(
            dimension_semantics=("arbitrary",),
        ),
    )(log_p, tail)

    samples = out[:, 0] - jnp.arange(n_rows, dtype=jnp.int32) * jnp.int32(n_cols)
    return samples.astype(jnp.int64)


# speedup vs baseline: 1.5214x; 1.0048x over previous
import functools

import jax
import jax.numpy as jnp
import numpy as np
from jax.experimental import pallas as pl
from jax.experimental.pallas import tpu as pltpu

_TINY = np.float32(np.finfo(np.float32).tiny)
_KS0 = np.uint32(0)
_KS1 = np.uint32(42)
_KS2 = np.uint32(42 ^ 0x1BD11BDA)

_C_BLK = 16384
_SLICE = 128


def _rotl(x, d):
    return (x << jnp.uint32(d)) | (x >> jnp.uint32(32 - d))


def _bits_from_x1(x):
    x0 = x
    x1 = _rotl(x, 13) ^ x
    for r in (15, 26, 6):
        x0 = x0 + x1
        x1 = _rotl(x1, r) ^ x0
    x0 = x0 + _KS1
    x1 = x1 + jnp.uint32(_KS2 + np.uint32(1))
    for r in (17, 29, 16, 24):
        x0 = x0 + x1
        x1 = _rotl(x1, r) ^ x0
    x0 = x0 + _KS2
    x1 = x1 + jnp.uint32(2)
    for r in (13, 15, 26, 6):
        x0 = x0 + x1
        x1 = _rotl(x1, r) ^ x0
    x1 = x1 + jnp.uint32(_KS1 + np.uint32(3))
    for r in (17, 29, 16, 24):
        x0 = x0 + x1
        x1 = _rotl(x1, r) ^ x0
    x0 = x0 + _KS1
    x1 = x1 + jnp.uint32(_KS2 + np.uint32(4))
    for r in (13, 15, 26, 6):
        x0 = x0 + x1
        x1 = _rotl(x1, r) ^ x0
    x0 = x0 + _KS2
    x1 = x1 + jnp.uint32(5)
    return x0 ^ x1


def _score_slice(lp, x1_in):
    bits = _bits_from_x1(x1_in)
    float_bits = (bits >> jnp.uint32(9)) | jnp.uint32(0x3F800000)
    frac = jax.lax.bitcast_convert_type(float_bits, jnp.float32) - jnp.float32(1.0)
    u = jnp.maximum(frac, _TINY)
    g = -jnp.log(-jnp.log(u))
    return lp + g


def _sample_kernel(logp_ref, tail_ref, out_ref, acc_val, acc_idx, *, n_rows,
                   n_cols, n_blocks, tail_blk):
    j = pl.program_id(0)
    n_slices = _C_BLK // _SLICE
    n_tail_slices = tail_blk // _SLICE

    pattern = (
        jax.lax.broadcasted_iota(jnp.uint32, (n_rows, _SLICE), 0)
        * jnp.uint32(n_cols)
        + jax.lax.broadcasted_iota(jnp.uint32, (n_rows, _SLICE), 1)
        + _KS1
    )

    @pl.when(j == 0)
    def _init():
        acc_val[...] = jnp.full((n_rows, _SLICE), -jnp.inf, jnp.float32)
        acc_idx[...] = jnp.zeros((n_rows, _SLICE), jnp.int32)

    base = j.astype(jnp.uint32) * jnp.uint32(_C_BLK)

    n_chains = 4
    per_chain = n_slices // n_chains
    chains = []
    for c in range(n_chains):
        cv = ci = None
        for s in range(c * per_chain, (c + 1) * per_chain):
            x1_in = pattern + (base + jnp.uint32(s * _SLICE))
            score = _score_slice(logp_ref[:, s * _SLICE:(s + 1) * _SLICE],
                                 x1_in)
            idx = x1_in.astype(jnp.int32)
            if cv is None:
                cv, ci = score, idx
            else:
                take = score > cv
                cv = jnp.where(take, score, cv)
                ci = jnp.where(take, idx, ci)
        chains.append((cv, ci))

    v = acc_val[...]
    ix = acc_idx[...]
    for cv, ci in chains:
        take = cv > v
        v = jnp.where(take, cv, v)
        ix = jnp.where(take, ci, ix)

    @pl.when(j < n_blocks - 1)
    def _store():
        acc_val[...] = v
        acc_idx[...] = ix

    @pl.when(j == n_blocks - 1)
    def _tail_and_finalize():
        vv, iixx = v, ix
        tail_base = jnp.uint32(n_blocks * _C_BLK)
        for s in range(n_tail_slices):
            x1_in = pattern + (tail_base + jnp.uint32(s * _SLICE))
            score = _score_slice(tail_ref[:, s * _SLICE:(s + 1) * _SLICE],
                                 x1_in)
            idx = x1_in.astype(jnp.int32)
            take = score > vv
            vv = jnp.where(take, score, vv)
            iixx = jnp.where(take, idx, iixx)
        row_max = jnp.max(vv, axis=1, keepdims=True)
        cand = jnp.where(vv >= row_max, iixx, jnp.int32(2**31 - 1))
        best = jnp.min(cand, axis=1, keepdims=True)
        out_ref[...] = jnp.broadcast_to(best - jnp.int32(int(_KS1)),
                                        (n_rows, _SLICE))


@jax.jit
def kernel(log_p):
    n_rows, n_cols = log_p.shape
    n_blocks = n_cols // _C_BLK
    tail_cols = n_cols - n_blocks * _C_BLK
    tail_blk = max(_SLICE, pl.cdiv(tail_cols, _SLICE) * _SLICE)

    tail = jnp.full((n_rows, tail_blk), -jnp.inf, jnp.float32)
    if tail_cols:
        tail = jax.lax.dynamic_update_slice(
            tail, log_p[:, n_blocks * _C_BLK:], (0, 0))

    body = functools.partial(
        _sample_kernel, n_rows=n_rows, n_cols=n_cols, n_blocks=n_blocks,
        tail_blk=tail_blk)

    out = pl.pallas_call(
        body,
        grid=(n_blocks,),
        in_specs=[
            pl.BlockSpec((n_rows, _C_BLK), lambda j: (0, j)),
            pl.BlockSpec((n_rows, tail_blk), lambda j: (0, 0)),
        ],
        out_specs=pl.BlockSpec((n_rows, _SLICE), lambda j: (0, 0)),
        out_shape=jax.ShapeDtypeStruct((n_rows, _SLICE), jnp.int32),
        scratch_shapes=[
            pltpu.VMEM((n_rows, _SLICE), jnp.float32),
            pltpu.VMEM((n_rows, _SLICE), jnp.int32),
        ],
        compiler_params=pltpu.CompilerParams(
            dimension_semantics=("arbitrary",),
        ),
    )(log_p, tail)

    samples = out[:, 0] - jnp.arange(n_rows, dtype=jnp.int32) * jnp.int32(n_cols)
    return samples.astype(jnp.int64)
